# Initial kernel scaffold; baseline (speedup 1.0000x reference)
import jax
import jax.numpy as jnp
from jax.experimental import pallas as pl


def kernel(x, sentence_index, Wq, bq, Wk, bk):
    raise NotImplementedError("write your pallas kernel here")


# speedup vs baseline: 9.8521x; 9.8521x over previous
import jax
import jax.numpy as jnp
from jax.experimental import pallas as pl
from jax.experimental.pallas import tpu as pltpu

_B, _S, _D = 8, 2048, 768
_NS = 64
_DQ = _D // 8


def _fused_body(seg_ref, x_ref, wq_ref, bq_ref, wk_ref, bk_ref, out_ref):
    xb = x_ref[0]
    seg_row = seg_ref[0]
    seg_col = seg_row.reshape(_S, 1)

    q = jnp.dot(xb, wq_ref[:], preferred_element_type=jnp.float32) + bq_ref[:]
    k = jnp.dot(xb, wk_ref[:], preferred_element_type=jnp.float32) + bk_ref[:]

    ids_ns_s = jax.lax.broadcasted_iota(jnp.int32, (_NS, _S), 0)
    ids_s_ns = jax.lax.broadcasted_iota(jnp.int32, (_S, _NS), 1)
    onehot = (ids_ns_s == seg_row).astype(jnp.float32)
    onehot_t = (ids_s_ns == seg_col).astype(jnp.float32)
    mask_t = ids_s_ns == seg_col

    ksum = jnp.dot(onehot, k, preferred_element_type=jnp.float32,
                   precision=jax.lax.Precision.HIGHEST)
    ksum_g = jnp.dot(onehot_t, ksum, preferred_element_type=jnp.float32,
                     precision=jax.lax.Precision.HIGHEST)
    s_col = jnp.sum(q * ksum_g, axis=1, keepdims=True)

    neg_inf = jnp.float32(-jnp.inf)
    smax = jnp.max(jnp.where(mask_t, s_col, neg_inf), axis=0, keepdims=True)
    smax_tok = jnp.sum(jnp.where(mask_t, smax, 0.0), axis=1, keepdims=True)
    e_col = jnp.exp(s_col - smax_tok)
    esum = jnp.sum(jnp.where(mask_t, e_col, 0.0), axis=0, keepdims=True)
    esum_tok = jnp.sum(jnp.where(mask_t, esum, 0.0), axis=1, keepdims=True)
    a_row = (e_col / esum_tok).reshape(1, _S)

    del onehot_t
    out_ref[0] = jnp.dot(onehot * a_row, xb, preferred_element_type=jnp.float32,
                         precision=jax.lax.Precision.HIGHEST)


def kernel(x, sentence_index, Wq, bq, Wk, bk):
    seg3 = sentence_index.astype(jnp.int32).reshape(_B, 1, _S)
    out = pl.pallas_call(
        _fused_body,
        grid=(_B,),
        in_specs=[
            pl.BlockSpec((1, 1, _S), lambda b: (b, 0, 0)),
            pl.BlockSpec((1, _S, _D), lambda b: (b, 0, 0)),
            pl.BlockSpec((_D, _DQ), lambda b: (0, 0)),
            pl.BlockSpec((1, _DQ), lambda b: (0, 0)),
            pl.BlockSpec((_D, _DQ), lambda b: (0, 0)),
            pl.BlockSpec((1, _DQ), lambda b: (0, 0)),
        ],
        out_specs=pl.BlockSpec((1, _NS, _D), lambda b: (b, 0, 0)),
        out_shape=jax.ShapeDtypeStruct((_B, _NS, _D), jnp.float32),
        compiler_params=pltpu.CompilerParams(
            dimension_semantics=("arbitrary",),
        ),
    )(seg3, x, Wq, bq.reshape(1, _DQ), Wk, bk.reshape(1, _DQ))
    return out[:, 1:]


# speedup vs baseline: 13.5341x; 1.3737x over previous
import jax
import jax.numpy as jnp
from jax.experimental import pallas as pl
from jax.experimental.pallas import tpu as pltpu

_B, _S, _D = 8, 2048, 768
_NS = 64
_DQ = _D // 8


def _fused_body(seg_ref, x_ref, wq_ref, bq_ref, wk_ref, bk_ref, out_ref):
    xb = x_ref[0]
    seg_row = seg_ref[0]
    seg_col = seg_row.reshape(_S, 1)

    q = jnp.dot(xb, wq_ref[:], preferred_element_type=jnp.float32) + bq_ref[:]
    k = jnp.dot(xb, wk_ref[:], preferred_element_type=jnp.float32) + bk_ref[:]

    ids_ns_s = jax.lax.broadcasted_iota(jnp.int32, (_NS, _S), 0)
    ids_s_ns = jax.lax.broadcasted_iota(jnp.int32, (_S, _NS), 1)
    onehot = (ids_ns_s == seg_row).astype(jnp.float32)
    onehot_t = (ids_s_ns == seg_col).astype(jnp.float32)
    mask_t = ids_s_ns == seg_col

    ksum = jnp.dot(onehot, k, preferred_element_type=jnp.float32,
                   precision=jax.lax.Precision.HIGHEST)
    ksum_g = jnp.dot(onehot_t, ksum, preferred_element_type=jnp.float32,
                     precision=jax.lax.Precision.HIGHEST)
    s_col = jnp.sum(q * ksum_g, axis=1, keepdims=True)

    neg_inf = jnp.float32(-jnp.inf)
    smax = jnp.max(jnp.where(mask_t, s_col, neg_inf), axis=0, keepdims=True)
    smax_tok = jnp.sum(jnp.where(mask_t, smax, 0.0), axis=1, keepdims=True)
    e_col = jnp.exp(s_col - smax_tok)
    esum = jnp.sum(jnp.where(mask_t, e_col, 0.0), axis=0, keepdims=True)
    esum_tok = jnp.sum(jnp.where(mask_t, esum, 0.0), axis=1, keepdims=True)
    a_row = (e_col / esum_tok).reshape(1, _S)

    del onehot_t
    out_ref[0] = jnp.dot(onehot * a_row, xb,
                         preferred_element_type=jnp.float32)


def kernel(x, sentence_index, Wq, bq, Wk, bk):
    seg3 = sentence_index.astype(jnp.int32).reshape(_B, 1, _S)
    out = pl.pallas_call(
        _fused_body,
        grid=(_B,),
        in_specs=[
            pl.BlockSpec((1, 1, _S), lambda b: (b, 0, 0)),
            pl.BlockSpec((1, _S, _D), lambda b: (b, 0, 0)),
            pl.BlockSpec((_D, _DQ), lambda b: (0, 0)),
            pl.BlockSpec((1, _DQ), lambda b: (0, 0)),
            pl.BlockSpec((_D, _DQ), lambda b: (0, 0)),
            pl.BlockSpec((1, _DQ), lambda b: (0, 0)),
        ],
        out_specs=pl.BlockSpec((1, _NS, _D), lambda b: (b, 0, 0)),
        out_shape=jax.ShapeDtypeStruct((_B, _NS, _D), jnp.float32),
        compiler_params=pltpu.CompilerParams(
            dimension_semantics=("arbitrary",),
        ),
    )(seg3, x, Wq, bq.reshape(1, _DQ), Wk, bk.reshape(1, _DQ))
    return out[:, 1:]
